# baseline (speedup 1.0000x reference)
import functools

import jax
import jax.numpy as jnp
from jax import lax
from jax.experimental import pallas as pl
from jax.experimental.pallas import tpu as pltpu
from jax.experimental.pallas import tpu_sc as plsc

NC = 2
NS = 16
NW = NC * NS
L = 16

B = 320000
D = 128
CH = 128
NCHUNK = B // CH
EUNROLL = 4


def _body(z_hbm, src_hbm, dst_hbm, out_hbm,
          idx_a, idx_b, rows_a, rows_b, out_v, tmp_v, sem_a, sem_b):
    wid = lax.axis_index("s") * NC + lax.axis_index("c")
    nj = (NCHUNK - wid + NW - 1) // NW
    lanes = lax.iota(jnp.int32, L)

    def chunk_body(j, carry):
        base = (wid + j * NW) * CH
        pltpu.sync_copy(src_hbm.at[pl.ds(base, CH)], idx_a)
        pltpu.sync_copy(dst_hbm.at[pl.ds(base, CH)], idx_b)
        ca = pltpu.async_copy(z_hbm.at[idx_a], rows_a, sem_a)
        cb = pltpu.async_copy(z_hbm.at[idx_b], rows_b, sem_b)
        ca.wait()
        cb.wait()

        lane15 = jnp.full((L,), L - 1, dtype=jnp.int32)

        def group_body(g, gcarry):
            for k in range(L):
                e = g * L + k
                acc0 = jnp.zeros((L,), jnp.float32)
                acc1 = jnp.zeros((L,), jnp.float32)
                for jf in range(D // L):
                    a = rows_a[e, pl.ds(jf * L, L)]
                    b = rows_b[e, pl.ds(jf * L, L)]
                    if jf % 2 == 0:
                        acc0 = acc0 + a * b
                    else:
                        acc1 = acc1 + a * b
                tmp_v[k] = plsc.cumsum(acc0 + acc1)
            res = plsc.load_gather(tmp_v, [lanes, lane15])
            plsc.store_scatter(out_v, [g * L + lanes], res)
            return gcarry

        lax.fori_loop(0, CH // L, group_body, None)
        pltpu.sync_copy(out_v, out_hbm.at[pl.ds(base, CH)])
        return carry

    lax.fori_loop(0, nj, chunk_body, None)


@functools.lru_cache(maxsize=None)
def _build():
    return pl.kernel(
        _body,
        out_type=jax.ShapeDtypeStruct((B,), jnp.float32),
        mesh=plsc.VectorSubcoreMesh(core_axis_name="c", subcore_axis_name="s"),
        compiler_params=pltpu.CompilerParams(needs_layout_passes=False),
        scratch_types=[
            pltpu.VMEM((CH,), jnp.int32),
            pltpu.VMEM((CH,), jnp.int32),
            pltpu.VMEM((CH, D), jnp.float32),
            pltpu.VMEM((CH, D), jnp.float32),
            pltpu.VMEM((CH,), jnp.float32),
            pltpu.VMEM((L, L), jnp.float32),
            pltpu.SemaphoreType.DMA,
            pltpu.SemaphoreType.DMA,
        ],
    )


@jax.jit
def kernel(z, edge_label_index):
    src = edge_label_index[0].astype(jnp.int32)
    dst = edge_label_index[1].astype(jnp.int32)
    return _build()(z, src, dst)


# speedup vs baseline: 1.9328x; 1.9328x over previous
import functools

import jax
import jax.numpy as jnp
from jax import lax
from jax.experimental import pallas as pl
from jax.experimental.pallas import tpu as pltpu
from jax.experimental.pallas import tpu_sc as plsc

NC = 2
NS = 16
NW = NC * NS
L = 16

B = 320000
D = 128
CH = 128
NCHUNK = B // CH


def _body(z_hbm, idx_hbm, out_hbm,
          idx0, idx1, rows_a0, rows_b0, rows_a1, rows_b1, out0, out1, tmp_v,
          isem0, isem1, asem0, asem1, bsem0, bsem1, osem0, osem1):
    idx2 = (idx0, idx1)
    rows_a = (rows_a0, rows_a1)
    rows_b = (rows_b0, rows_b1)
    out_v = (out0, out1)
    isem = (isem0, isem1)
    asem = (asem0, asem1)
    bsem = (bsem0, bsem1)
    osem = (osem0, osem1)

    wid = lax.axis_index("s") * NC + lax.axis_index("c")
    nj = (NCHUNK - wid + NW - 1) // NW
    lanes = lax.iota(jnp.int32, L)
    lane15 = jnp.full((L,), L - 1, dtype=jnp.int32)

    def chunk_of(j):
        return wid + j * NW

    def issue_gathers(s):
        pltpu.async_copy(z_hbm.at[idx2[s].at[0, 0]], rows_a[s], asem[s])
        pltpu.async_copy(z_hbm.at[idx2[s].at[0, 1]], rows_b[s], bsem[s])

    def wait_gathers(s):
        pltpu.make_async_copy(z_hbm.at[idx2[s].at[0, 0]], rows_a[s], asem[s]).wait()
        pltpu.make_async_copy(z_hbm.at[idx2[s].at[0, 1]], rows_b[s], bsem[s]).wait()

    def wait_idx(s):
        pltpu.make_async_copy(idx_hbm.at[pl.ds(0, 1)], idx2[s], isem[s]).wait()

    def wait_out(s):
        pltpu.make_async_copy(out_v[s], out_hbm.at[pl.ds(0, CH)], osem[s]).wait()

    def compute(s):
        ra, rb, ov = rows_a[s], rows_b[s], out_v[s]

        def group_body(g, gcarry):
            for k in range(L):
                e = g * L + k
                acc0 = jnp.zeros((L,), jnp.float32)
                acc1 = jnp.zeros((L,), jnp.float32)
                for jf in range(D // L):
                    a = ra[e, pl.ds(jf * L, L)]
                    b = rb[e, pl.ds(jf * L, L)]
                    if jf % 2 == 0:
                        acc0 = acc0 + a * b
                    else:
                        acc1 = acc1 + a * b
                tmp_v[k] = plsc.cumsum(acc0 + acc1)
            res = plsc.load_gather(tmp_v, [lanes, lane15])
            plsc.store_scatter(ov, [g * L + lanes], res)
            return gcarry

        lax.fori_loop(0, CH // L, group_body, None)

    pltpu.sync_copy(idx_hbm.at[pl.ds(chunk_of(0), 1)], idx2[0])
    issue_gathers(0)

    @pl.when(nj > 1)
    def _():
        pltpu.async_copy(idx_hbm.at[pl.ds(chunk_of(1), 1)], idx2[1], isem[1])

    npairs = (nj + 1) // 2

    def pair_body(p, carry):
        for s in (0, 1):
            j = 2 * p + s
            o = 1 - s

            @pl.when(j < nj)
            def _process():
                @pl.when(j + 1 < nj)
                def _():
                    wait_idx(o)
                    issue_gathers(o)

                wait_gathers(s)

                @pl.when(j + 2 < nj)
                def _():
                    pltpu.async_copy(idx_hbm.at[pl.ds(chunk_of(j + 2), 1)],
                                     idx2[s], isem[s])

                @pl.when(j >= 2)
                def _():
                    wait_out(s)

                compute(s)
                pltpu.async_copy(out_v[s],
                                 out_hbm.at[pl.ds(chunk_of(j) * CH, CH)],
                                 osem[s])

        return carry

    lax.fori_loop(0, npairs, pair_body, None)

    wait_out(0)
    wait_out(1)


@functools.lru_cache(maxsize=None)
def _build():
    return pl.kernel(
        _body,
        out_type=jax.ShapeDtypeStruct((B,), jnp.float32),
        mesh=plsc.VectorSubcoreMesh(core_axis_name="c", subcore_axis_name="s"),
        compiler_params=pltpu.CompilerParams(needs_layout_passes=False),
        scratch_types=[
            pltpu.VMEM((1, 2, CH), jnp.int32),
            pltpu.VMEM((1, 2, CH), jnp.int32),
            pltpu.VMEM((CH, D), jnp.float32),
            pltpu.VMEM((CH, D), jnp.float32),
            pltpu.VMEM((CH, D), jnp.float32),
            pltpu.VMEM((CH, D), jnp.float32),
            pltpu.VMEM((CH,), jnp.float32),
            pltpu.VMEM((CH,), jnp.float32),
            pltpu.VMEM((L, L), jnp.float32),
            pltpu.SemaphoreType.DMA,
            pltpu.SemaphoreType.DMA,
            pltpu.SemaphoreType.DMA,
            pltpu.SemaphoreType.DMA,
            pltpu.SemaphoreType.DMA,
            pltpu.SemaphoreType.DMA,
            pltpu.SemaphoreType.DMA,
            pltpu.SemaphoreType.DMA,
        ],
    )


@jax.jit
def kernel(z, edge_label_index):
    idx = edge_label_index.astype(jnp.int32)
    idx_pack = idx.reshape(2, NCHUNK, CH).transpose(1, 0, 2)
    return _build()(z, idx_pack)
